# Initial kernel scaffold; baseline (speedup 1.0000x reference)
import jax
import jax.numpy as jnp
from jax.experimental import pallas as pl


def kernel(graph1, feat1, graph2, feat2, graph, feat, W0, b0, W1, b1, W2, b2):
    raise NotImplementedError("write your pallas kernel here")


# speedup vs baseline: 1.8661x; 1.8661x over previous
import functools

import jax
import jax.numpy as jnp
from jax import lax
from jax.experimental import pallas as pl
from jax.experimental.pallas import tpu as pltpu
from jax.experimental.pallas import tpu_sc as plsc

N_NODES = 10000
N_EDGES = 160000
N_ROWS = 10240
E_PAD = 163840
EDGES_PER_TILE = E_PAD // 16
BATCH = 128
N_BATCH = EDGES_PER_TILE // BATCH
ROWS_PER_TILE = N_ROWS // 16
CW = 128

_mesh = plsc.VectorSubcoreMesh(core_axis_name="c", subcore_axis_name="s")


def _deg_body(src_hbm, dst_hbm, ones_hbm, zeros_hbm, out_hbm,
              idx_v, ones_v, acc_sh):
    c = lax.axis_index("c")
    s = lax.axis_index("s")
    pltpu.sync_copy(ones_hbm, ones_v)
    pltpu.sync_copy(zeros_hbm, acc_sh.at[pl.ds(s * ROWS_PER_TILE, ROWS_PER_TILE)])
    plsc.subcore_barrier()

    def count(edge_hbm):
        def body(j, carry):
            off = s * EDGES_PER_TILE + j * BATCH
            pltpu.sync_copy(edge_hbm.at[pl.ds(off, BATCH)], idx_v)
            pltpu.sync_copy(ones_v, acc_sh.at[idx_v], add=True)
            return carry
        lax.fori_loop(0, N_BATCH, body, 0)

    @pl.when(c == 0)
    def _():
        count(src_hbm)

    @pl.when(c == 1)
    def _():
        count(dst_hbm)

    plsc.subcore_barrier()
    pltpu.sync_copy(acc_sh.at[pl.ds(s * ROWS_PER_TILE, ROWS_PER_TILE)],
                    out_hbm.at[c, pl.ds(s * ROWS_PER_TILE, ROWS_PER_TILE)])


_deg_call = pl.kernel(
    _deg_body,
    out_type=jax.ShapeDtypeStruct((2, N_ROWS), jnp.float32),
    mesh=_mesh,
    scratch_types=[
        pltpu.VMEM((BATCH,), jnp.int32),
        pltpu.VMEM((BATCH,), jnp.float32),
        pltpu.VMEM_SHARED((N_ROWS,), jnp.float32),
    ],
)


def _agg_body(x0_hbm, x1_hbm, src_hbm, dst_hbm, zeros_hbm, out_hbm,
              src_v, dst_v, rows_v, acc_sh, sem):
    c = lax.axis_index("c")
    s = lax.axis_index("s")
    pltpu.sync_copy(zeros_hbm, acc_sh.at[pl.ds(s * ROWS_PER_TILE, ROWS_PER_TILE)])
    plsc.subcore_barrier()

    def run(x_hbm):
        def body(j, carry):
            off = s * EDGES_PER_TILE + j * BATCH
            pltpu.sync_copy(src_hbm.at[pl.ds(off, BATCH)], src_v)
            pltpu.sync_copy(dst_hbm.at[pl.ds(off, BATCH)], dst_v)
            pltpu.async_copy(x_hbm.at[src_v], rows_v, sem).wait()
            pltpu.sync_copy(rows_v, acc_sh.at[dst_v], add=True)
            return carry
        lax.fori_loop(0, N_BATCH, body, 0)

    @pl.when(c == 0)
    def _():
        run(x0_hbm)

    @pl.when(c == 1)
    def _():
        run(x1_hbm)

    plsc.subcore_barrier()
    pltpu.sync_copy(acc_sh.at[pl.ds(s * ROWS_PER_TILE, ROWS_PER_TILE)],
                    out_hbm.at[c, pl.ds(s * ROWS_PER_TILE, ROWS_PER_TILE)])


_agg_call = pl.kernel(
    _agg_body,
    out_type=jax.ShapeDtypeStruct((2, N_ROWS, CW), jnp.float32),
    mesh=_mesh,
    scratch_types=[
        pltpu.VMEM((BATCH,), jnp.int32),
        pltpu.VMEM((BATCH,), jnp.int32),
        pltpu.VMEM((BATCH, CW), jnp.float32),
        pltpu.VMEM_SHARED((N_ROWS, CW), jnp.float32),
        pltpu.SemaphoreType.DMA,
    ],
)


def _agg(x, src, dst, zeros1d_unused=None):
    d = x.shape[1]
    npairs = d // (2 * CW)
    xt = x.reshape(N_ROWS, npairs, 2, CW).transpose(1, 2, 0, 3)
    zeros2d = jnp.zeros((ROWS_PER_TILE, CW), jnp.float32)
    parts = []
    for i in range(npairs):
        p = _agg_call(xt[i, 0], xt[i, 1], src, dst, zeros2d)
        parts.append(p[0])
        parts.append(p[1])
    return jnp.concatenate(parts, axis=1)


def _scales_kernel(cnt_ref, o_ref):
    o_ref[...] = lax.rsqrt(jnp.maximum(cnt_ref[...], 1.0))


def _scales(counts):
    return pl.pallas_call(
        _scales_kernel,
        out_shape=jax.ShapeDtypeStruct((2, N_ROWS), jnp.float32),
    )(counts)


def _rowscale_kernel(x_ref, s_ref, o_ref):
    o_ref[...] = x_ref[...] * s_ref[...]


def _rowscale(x, s_col):
    m, d = x.shape
    bm = 2048
    return pl.pallas_call(
        _rowscale_kernel,
        grid=(m // bm,),
        in_specs=[
            pl.BlockSpec((bm, d), lambda i: (i, 0)),
            pl.BlockSpec((bm, 1), lambda i: (i, 0)),
        ],
        out_specs=pl.BlockSpec((bm, d), lambda i: (i, 0)),
        out_shape=jax.ShapeDtypeStruct((m, d), jnp.float32),
    )(x, s_col)


def _mm_kernel(a_ref, w_ref, t_ref, s_ref, b_ref, o_ref):
    a = a_ref[...] * t_ref[...]
    acc = jnp.dot(a, w_ref[...], preferred_element_type=jnp.float32)
    o_ref[...] = jnp.maximum(acc + b_ref[...], 0.0) * s_ref[...]


def _mm_plain_kernel(a_ref, w_ref, o_ref):
    o_ref[...] = jnp.dot(a_ref[...], w_ref[...],
                         preferred_element_type=jnp.float32)


def _mm_fused(a, w, t_col, s_col, b):
    m, k = a.shape
    n = w.shape[1]
    bm, bn = 2048, min(n, 1024)
    return pl.pallas_call(
        _mm_kernel,
        grid=(m // bm, n // bn),
        in_specs=[
            pl.BlockSpec((bm, k), lambda i, j: (i, 0)),
            pl.BlockSpec((k, bn), lambda i, j: (0, j)),
            pl.BlockSpec((bm, 1), lambda i, j: (i, 0)),
            pl.BlockSpec((bm, 1), lambda i, j: (i, 0)),
            pl.BlockSpec((1, bn), lambda i, j: (0, j)),
        ],
        out_specs=pl.BlockSpec((bm, bn), lambda i, j: (i, j)),
        out_shape=jax.ShapeDtypeStruct((m, n), jnp.float32),
    )(a, w, t_col, s_col, b.reshape(1, n))


def _mm_plain(a, w):
    m, k = a.shape
    n = w.shape[1]
    bm, bn = 2048, min(n, 1024)
    return pl.pallas_call(
        _mm_plain_kernel,
        grid=(m // bm, n // bn),
        in_specs=[
            pl.BlockSpec((bm, k), lambda i, j: (i, 0)),
            pl.BlockSpec((k, bn), lambda i, j: (0, j)),
        ],
        out_specs=pl.BlockSpec((bm, bn), lambda i, j: (i, j)),
        out_shape=jax.ShapeDtypeStruct((m, n), jnp.float32),
    )(a, w)


def _bias_scale_kernel(a_ref, t_ref, b_ref, o_ref):
    o_ref[...] = a_ref[...] * t_ref[...] + b_ref[...]


def _bias_scale(a, t_col, b):
    m, n = a.shape
    bm = 2048
    return pl.pallas_call(
        _bias_scale_kernel,
        grid=(m // bm,),
        in_specs=[
            pl.BlockSpec((bm, n), lambda i: (i, 0)),
            pl.BlockSpec((bm, 1), lambda i: (i, 0)),
            pl.BlockSpec((1, n), lambda i: (0, 0)),
        ],
        out_specs=pl.BlockSpec((bm, n), lambda i: (i, 0)),
        out_shape=jax.ShapeDtypeStruct((m, n), jnp.float32),
    )(a, t_col, b.reshape(1, n))


def _encoder(g, x, W0, b0, W1, b1, W2, b2):
    src = jnp.concatenate(
        [g[0].astype(jnp.int32),
         jnp.full((E_PAD - N_EDGES,), N_NODES, jnp.int32)])
    dst = jnp.concatenate(
        [g[1].astype(jnp.int32),
         jnp.full((E_PAD - N_EDGES,), N_NODES, jnp.int32)])

    ones = jnp.ones((BATCH,), jnp.float32)
    zeros1d = jnp.zeros((ROWS_PER_TILE,), jnp.float32)
    counts = _deg_call(src, dst, ones, zeros1d)
    scales = _scales(counts)
    s_col = scales[0].reshape(N_ROWS, 1)
    t_col = scales[1].reshape(N_ROWS, 1)

    x_p = jnp.pad(x, ((0, N_ROWS - N_NODES), (0, 0)))
    u = _rowscale(x_p, s_col)
    a1 = _agg(u, src, dst)
    v = _mm_fused(a1, W0, t_col, s_col, b0)
    a2 = _agg(v, src, dst)
    h = _mm_fused(a2, W1, t_col, s_col, b1)
    w = _mm_plain(h, W2)
    a3 = _agg(w, src, dst)
    z = _bias_scale(a3, t_col, b2)
    return z[:N_NODES]


@jax.jit
def kernel(graph1, feat1, graph2, feat2, graph, feat,
           W0, b0, W1, b1, W2, b2):
    z1 = _encoder(graph1, feat1, W0, b0, W1, b1, W2, b2)
    z2 = _encoder(graph2, feat2, W0, b0, W1, b1, W2, b2)
    z = _encoder(graph, feat, W0, b0, W1, b1, W2, b2)
    return (z1, z2, z)


# speedup vs baseline: 2.2796x; 1.2216x over previous
import functools

import jax
import jax.numpy as jnp
from jax import lax
from jax.experimental import pallas as pl
from jax.experimental.pallas import tpu as pltpu
from jax.experimental.pallas import tpu_sc as plsc

N_NODES = 10000
N_EDGES = 160000
N_ROWS = 10240
E_PAD = 163840
EDGES_PER_TILE = E_PAD // 16
BATCH = 128
N_BATCH = EDGES_PER_TILE // BATCH
ROWS_PER_TILE = N_ROWS // 16
CW = 128

_mesh = plsc.VectorSubcoreMesh(core_axis_name="c", subcore_axis_name="s")


def _deg_body(src_hbm, dst_hbm, ones_hbm, zeros_hbm, out_hbm,
              idx_all, ones_v, acc_sh, sem):
    c = lax.axis_index("c")
    s = lax.axis_index("s")
    pltpu.sync_copy(ones_hbm, ones_v)
    pltpu.sync_copy(zeros_hbm, acc_sh.at[pl.ds(s * ROWS_PER_TILE, ROWS_PER_TILE)])

    @pl.when(c == 0)
    def _():
        pltpu.sync_copy(src_hbm.at[s], idx_all)

    @pl.when(c == 1)
    def _():
        pltpu.sync_copy(dst_hbm.at[s], idx_all)

    plsc.subcore_barrier()

    def group(g, carry):
        descs = []
        for b in range(8):
            descs.append(pltpu.async_copy(
                ones_v, acc_sh.at[idx_all.at[g * 8 + b]], sem, add=True))
        for d in descs:
            d.wait()
        return carry
    lax.fori_loop(0, N_BATCH // 8, group, 0)

    plsc.subcore_barrier()
    pltpu.sync_copy(acc_sh.at[pl.ds(s * ROWS_PER_TILE, ROWS_PER_TILE)],
                    out_hbm.at[c, pl.ds(s * ROWS_PER_TILE, ROWS_PER_TILE)])


_deg_call = pl.kernel(
    _deg_body,
    out_type=jax.ShapeDtypeStruct((2, N_ROWS), jnp.float32),
    mesh=_mesh,
    scratch_types=[
        pltpu.VMEM((N_BATCH, BATCH), jnp.int32),
        pltpu.VMEM((BATCH,), jnp.float32),
        pltpu.VMEM_SHARED((N_ROWS,), jnp.float32),
        pltpu.SemaphoreType.DMA,
    ],
)


_NBUF = 2
_CHUNK = 16


def _agg_body(x0_hbm, x1_hbm, src_hbm, dst_hbm, zeros_hbm, out_hbm,
              src_ch, dst_ch, buf0, buf1, acc_sh,
              gsem0, gsem1, ssem):
    c = lax.axis_index("c")
    s = lax.axis_index("s")
    bufs = (buf0, buf1)
    gsems = (gsem0, gsem1)
    pltpu.sync_copy(zeros_hbm, acc_sh.at[pl.ds(s * ROWS_PER_TILE, ROWS_PER_TILE)])
    plsc.subcore_barrier()

    def run(x_hbm):
        def super_body(sg, carry):
            pltpu.sync_copy(src_hbm.at[s, pl.ds(sg * _CHUNK, _CHUNK)], src_ch)
            pltpu.sync_copy(dst_hbm.at[s, pl.ds(sg * _CHUNK, _CHUNK)], dst_ch)

            def group(g, carry2):
                gd = []
                for b in range(_NBUF):
                    gd.append(pltpu.async_copy(
                        x_hbm.at[src_ch.at[g * _NBUF + b]], bufs[b], gsems[b]))
                sd = []
                for b in range(_NBUF):
                    gd[b].wait()
                    sd.append(pltpu.async_copy(
                        bufs[b], acc_sh.at[dst_ch.at[g * _NBUF + b]], ssem,
                        add=True))
                for d in sd:
                    d.wait()
                return carry2
            lax.fori_loop(0, _CHUNK // _NBUF, group, 0)
            return carry
        lax.fori_loop(0, N_BATCH // _CHUNK, super_body, 0)

    @pl.when(c == 0)
    def _():
        run(x0_hbm)

    @pl.when(c == 1)
    def _():
        run(x1_hbm)

    plsc.subcore_barrier()
    pltpu.sync_copy(acc_sh.at[pl.ds(s * ROWS_PER_TILE, ROWS_PER_TILE)],
                    out_hbm.at[c, pl.ds(s * ROWS_PER_TILE, ROWS_PER_TILE)])


_agg_call = pl.kernel(
    _agg_body,
    out_type=jax.ShapeDtypeStruct((2, N_ROWS, CW), jnp.float32),
    mesh=_mesh,
    scratch_types=[
        pltpu.VMEM((_CHUNK, BATCH), jnp.int32),
        pltpu.VMEM((_CHUNK, BATCH), jnp.int32),
        pltpu.VMEM((BATCH, CW), jnp.float32),
        pltpu.VMEM((BATCH, CW), jnp.float32),
        pltpu.VMEM_SHARED((N_ROWS, CW), jnp.float32),
        pltpu.SemaphoreType.DMA,
        pltpu.SemaphoreType.DMA,
        pltpu.SemaphoreType.DMA,
    ],
)


def _agg(x, src, dst, zeros1d_unused=None):
    d = x.shape[1]
    npairs = d // (2 * CW)
    xt = x.reshape(N_ROWS, npairs, 2, CW).transpose(1, 2, 0, 3)
    zeros2d = jnp.zeros((ROWS_PER_TILE, CW), jnp.float32)
    parts = []
    for i in range(npairs):
        p = _agg_call(xt[i, 0], xt[i, 1], src, dst, zeros2d)
        parts.append(p[0])
        parts.append(p[1])
    return jnp.concatenate(parts, axis=1)


def _scales_kernel(cnt_ref, o_ref):
    o_ref[...] = lax.rsqrt(jnp.maximum(cnt_ref[...], 1.0))


def _scales(counts):
    return pl.pallas_call(
        _scales_kernel,
        out_shape=jax.ShapeDtypeStruct((2, N_ROWS), jnp.float32),
    )(counts)


def _rowscale_kernel(x_ref, s_ref, o_ref):
    o_ref[...] = x_ref[...] * s_ref[...]


def _rowscale(x, s_col):
    m, d = x.shape
    bm = 2048
    return pl.pallas_call(
        _rowscale_kernel,
        grid=(m // bm,),
        in_specs=[
            pl.BlockSpec((bm, d), lambda i: (i, 0)),
            pl.BlockSpec((bm, 1), lambda i: (i, 0)),
        ],
        out_specs=pl.BlockSpec((bm, d), lambda i: (i, 0)),
        out_shape=jax.ShapeDtypeStruct((m, d), jnp.float32),
    )(x, s_col)


def _mm_kernel(a_ref, w_ref, t_ref, s_ref, b_ref, o_ref):
    a = a_ref[...] * t_ref[...]
    acc = jnp.dot(a, w_ref[...], preferred_element_type=jnp.float32)
    o_ref[...] = jnp.maximum(acc + b_ref[...], 0.0) * s_ref[...]


def _mm_plain_kernel(a_ref, w_ref, o_ref):
    o_ref[...] = jnp.dot(a_ref[...], w_ref[...],
                         preferred_element_type=jnp.float32)


def _mm_fused(a, w, t_col, s_col, b):
    m, k = a.shape
    n = w.shape[1]
    bm, bn = 2048, min(n, 1024)
    return pl.pallas_call(
        _mm_kernel,
        grid=(m // bm, n // bn),
        in_specs=[
            pl.BlockSpec((bm, k), lambda i, j: (i, 0)),
            pl.BlockSpec((k, bn), lambda i, j: (0, j)),
            pl.BlockSpec((bm, 1), lambda i, j: (i, 0)),
            pl.BlockSpec((bm, 1), lambda i, j: (i, 0)),
            pl.BlockSpec((1, bn), lambda i, j: (0, j)),
        ],
        out_specs=pl.BlockSpec((bm, bn), lambda i, j: (i, j)),
        out_shape=jax.ShapeDtypeStruct((m, n), jnp.float32),
    )(a, w, t_col, s_col, b.reshape(1, n))


def _mm_plain(a, w):
    m, k = a.shape
    n = w.shape[1]
    bm, bn = 2048, min(n, 1024)
    return pl.pallas_call(
        _mm_plain_kernel,
        grid=(m // bm, n // bn),
        in_specs=[
            pl.BlockSpec((bm, k), lambda i, j: (i, 0)),
            pl.BlockSpec((k, bn), lambda i, j: (0, j)),
        ],
        out_specs=pl.BlockSpec((bm, bn), lambda i, j: (i, j)),
        out_shape=jax.ShapeDtypeStruct((m, n), jnp.float32),
    )(a, w)


def _bias_scale_kernel(a_ref, t_ref, b_ref, o_ref):
    o_ref[...] = a_ref[...] * t_ref[...] + b_ref[...]


def _bias_scale(a, t_col, b):
    m, n = a.shape
    bm = 2048
    return pl.pallas_call(
        _bias_scale_kernel,
        grid=(m // bm,),
        in_specs=[
            pl.BlockSpec((bm, n), lambda i: (i, 0)),
            pl.BlockSpec((bm, 1), lambda i: (i, 0)),
            pl.BlockSpec((1, n), lambda i: (0, 0)),
        ],
        out_specs=pl.BlockSpec((bm, n), lambda i: (i, 0)),
        out_shape=jax.ShapeDtypeStruct((m, n), jnp.float32),
    )(a, t_col, b.reshape(1, n))


def _encoder(g, x, W0, b0, W1, b1, W2, b2):
    src = jnp.concatenate(
        [g[0].astype(jnp.int32),
         jnp.full((E_PAD - N_EDGES,), N_NODES, jnp.int32)]
    ).reshape(16, N_BATCH, BATCH)
    dst = jnp.concatenate(
        [g[1].astype(jnp.int32),
         jnp.full((E_PAD - N_EDGES,), N_NODES, jnp.int32)]
    ).reshape(16, N_BATCH, BATCH)

    ones = jnp.ones((BATCH,), jnp.float32)
    zeros1d = jnp.zeros((ROWS_PER_TILE,), jnp.float32)
    counts = _deg_call(src, dst, ones, zeros1d)
    scales = _scales(counts)
    s_col = scales[0].reshape(N_ROWS, 1)
    t_col = scales[1].reshape(N_ROWS, 1)

    x_p = jnp.pad(x, ((0, N_ROWS - N_NODES), (0, 0)))
    u = _rowscale(x_p, s_col)
    a1 = _agg(u, src, dst)
    v = _mm_fused(a1, W0, t_col, s_col, b0)
    a2 = _agg(v, src, dst)
    h = _mm_fused(a2, W1, t_col, s_col, b1)
    w = _mm_plain(h, W2)
    a3 = _agg(w, src, dst)
    z = _bias_scale(a3, t_col, b2)
    return z[:N_NODES]


@jax.jit
def kernel(graph1, feat1, graph2, feat2, graph, feat,
           W0, b0, W1, b1, W2, b2):
    z1 = _encoder(graph1, feat1, W0, b0, W1, b1, W2, b2)
    z2 = _encoder(graph2, feat2, W0, b0, W1, b1, W2, b2)
    z = _encoder(graph, feat, W0, b0, W1, b1, W2, b2)
    return (z1, z2, z)
